# baseline (speedup 1.0000x reference)
import functools
import math

import jax
import jax.numpy as jnp
from jax import lax
from jax.experimental import pallas as pl
from jax.experimental.pallas import tpu as pltpu
from jax.experimental.pallas import tpu_sc as plsc

_K = 10
_POS_PENALTY = 1.2
_NW = 32
_SC_ROWS = 1024
_ROW_BLOCK = 256
_W = [1.0 / math.log2(j + 2.0) for j in range(_K)]
_NEG_BIG = float(-3.4028235e38)

_NET8 = [
    (0, 1), (2, 3), (4, 5), (6, 7),
    (0, 2), (1, 3), (4, 6), (5, 7),
    (1, 2), (5, 6), (0, 4), (1, 5),
    (2, 6), (3, 7), (2, 4), (3, 5),
    (1, 2), (3, 4), (5, 6),
]


def _sort_dir(v, descending):
    return plsc.sort_key_val(v, v, descending=descending)[0]


def _top16_tree(vs, descending=True):
    if len(vs) == 1:
        return _sort_dir(vs[0], descending)
    mid = len(vs) // 2
    left = _top16_tree(vs[:mid], descending=False)
    right = _top16_tree(vs[mid:], descending=True)
    return _sort_dir(jnp.maximum(left, right), descending)


def _row_top16(rows_v, off, nv, rem, lane, neginf):
    vs = [rows_v[pl.ds(off + i * 16, 16)] for i in range(nv - 1)]
    vlast = rows_v[pl.ds(off + (nv - 1) * 16, 16)]
    vs.append(jnp.where(lane < rem, vlast, neginf))
    return _top16_tree(vs, descending=True)


def _make_sc_topk(s_rows, n):
    rpw = s_rows // _NW
    chunk = rpw if rpw <= 32 else rpw // ((rpw + 31) // 32)
    nchunk = rpw // chunk
    assert chunk * nchunk == rpw
    nv = (n + 15) // 16
    padrow = nv * 16
    rem = n - (nv - 1) * 16
    mesh = plsc.VectorSubcoreMesh(core_axis_name="c", subcore_axis_name="s")

    @functools.partial(
        pl.kernel,
        out_type=jax.ShapeDtypeStruct((s_rows * 16,), jnp.float32),
        mesh=mesh,
        scratch_types=[
            pltpu.VMEM((chunk * padrow,), jnp.float32),
            pltpu.VMEM((chunk * padrow,), jnp.float32),
            pltpu.VMEM((chunk * 16,), jnp.float32),
            pltpu.SemaphoreType.DMA,
            pltpu.SemaphoreType.DMA,
        ],
        compiler_params=pltpu.CompilerParams(needs_layout_passes=False),
    )
    def sc_topk(neg_hbm, out_hbm, rows_a, rows_b, top_v2, sem_a, sem_b):
        wid = lax.axis_index("s") * 2 + lax.axis_index("c")
        base = wid * rpw
        lane = lax.iota(jnp.int32, 16)
        neginf = jnp.full((16,), -jnp.inf, jnp.float32)
        bufs = [rows_a, rows_b]
        sems = [sem_a, sem_b]

        def issue(ci, buf, sem):
            return [
                pltpu.async_copy(
                    neg_hbm.at[pl.ds((base + ci * chunk + r) * n, n)],
                    buf.at[pl.ds(r * padrow, n)],
                    sem,
                )
                for r in range(chunk)
            ]

        pend = issue(0, bufs[0], sems[0])
        for ci in range(nchunk):
            for d in pend:
                d.wait()
            if ci + 1 < nchunk:
                pend = issue(ci + 1, bufs[(ci + 1) % 2], sems[(ci + 1) % 2])
            buf = bufs[ci % 2]

            def row_body(r2, carry, buf=buf):
                for u in range(2):
                    r = r2 * 2 + u
                    top_v2[pl.ds(r * 16, 16)] = _row_top16(
                        buf, r * padrow, nv, rem, lane, neginf
                    )
                return carry

            lax.fori_loop(0, chunk // 2, row_body, 0)
            pltpu.sync_copy(
                top_v2,
                out_hbm.at[pl.ds((base + ci * chunk) * 16, chunk * 16)],
            )

    return sc_topk


def _tc_topk_loss(x, pos):
    r, c = x.shape
    nq = c // 128
    lane = jax.lax.broadcasted_iota(jnp.int32, (r, 128), 1)
    xs = [x[:, i * 128:(i + 1) * 128] for i in range(nq)]
    s = []
    for i in range(nq):
        bits = jax.lax.bitcast_convert_type(xs[i], jnp.int32)
        pk = (bits & ~jnp.int32(1023)) | ((c - 1 - i * 128) - lane)
        s.append(jax.lax.bitcast_convert_type(pk, jnp.float32))
    for a, b in _NET8:
        hi = jnp.maximum(s[a], s[b])
        lo = jnp.minimum(s[a], s[b])
        s[a], s[b] = hi, lo
    tm = jnp.full((r, 128), _NEG_BIG, jnp.float32)
    m0p = None
    for j in range(_K):
        km = jnp.max(s[0], axis=1, keepdims=True)
        if j == 0:
            m0p = km
        qmax = min(nq - 1, _K - 1 - j)
        if qmax > 0:
            pop = s[0] == km
            for q in range(qmax):
                s[q] = jnp.where(pop, s[q + 1], s[q])
            if j + 8 < _K:
                s[nq - 1] = jnp.where(pop, _NEG_BIG, s[nq - 1])
        tm = jnp.where(lane == j, km, tm)
    tb = jax.lax.bitcast_convert_type(tm, jnp.int32) & ~jnp.int32(1023)
    tm = jax.lax.bitcast_convert_type(tb, jnp.float32)
    m0b = jax.lax.bitcast_convert_type(m0p, jnp.int32) & ~jnp.int32(1023)
    m0 = jax.lax.bitcast_convert_type(m0b, jnp.float32)
    wv = jnp.zeros((1, 128), jnp.float32)
    lane1 = jax.lax.broadcasted_iota(jnp.int32, (1, 128), 1)
    for j in range(_K):
        wv = wv + _W[j] * jnp.where(lane1 == j, 1.0, 0.0)
    z = tm - pos
    sp = jnp.maximum(z, 0.0) + jnp.log1p(jnp.exp(-jnp.abs(z)))
    pen = jnp.where(pos < m0, _POS_PENALTY, 1.0)
    return jnp.sum(jnp.sum(sp * wv, axis=1, keepdims=True) * pen)


def _pad_lanes(x):
    r, n = x.shape
    c = ((n + 127) // 128) * 128
    if c == n:
        return x
    return jnp.concatenate(
        [x, jnp.full((r, c - n), _NEG_BIG, jnp.float32)], axis=1
    )


def _tc_body(pos_ref, neg_ref, out_ref):
    blk = _tc_topk_loss(_pad_lanes(neg_ref[...]), pos_ref[...])

    @pl.when(pl.program_id(0) == 0)
    def _():
        out_ref[0, 0] = 0.0

    out_ref[0, 0] += blk


def _sc_loss_body(pos_lo_ref, topk_ref, out_ref):
    x = topk_ref[...]
    pos = pos_lo_ref[...]
    b, c = x.shape
    jl = lax.broadcasted_iota(jnp.int32, (b, c), 1)
    wv = jnp.where(
        jl < _K, 1.0 / (jnp.log2(jl.astype(jnp.float32) + 2.0)), 0.0
    )
    z = x - pos
    sp = jnp.maximum(z, 0.0) + jnp.log1p(jnp.exp(-jnp.abs(z)))
    m0 = jnp.max(x, axis=1, keepdims=True)
    pen = jnp.where(pos < m0, _POS_PENALTY, 1.0)
    out_ref[0, 0] = jnp.sum(sp * wv * pen)


def kernel(pos, neg):
    if pos.ndim == 1:
        pos = pos[:, None]
    b, n = neg.shape
    k = min(_K, n)
    s_rows = _SC_ROWS
    off = s_rows // _ROW_BLOCK

    topk = _make_sc_topk(s_rows, n)(
        neg[:s_rows].reshape(-1)
    ).reshape(s_rows, 16)

    grid = (b - s_rows) // _ROW_BLOCK
    tc_sum = pl.pallas_call(
        _tc_body,
        grid=(grid,),
        in_specs=[
            pl.BlockSpec((_ROW_BLOCK, 1), lambda i: (i + off, 0)),
            pl.BlockSpec((_ROW_BLOCK, n), lambda i: (i + off, 0)),
        ],
        out_specs=pl.BlockSpec(memory_space=pltpu.SMEM),
        out_shape=jax.ShapeDtypeStruct((1, 1), jnp.float32),
    )(pos, neg)

    sc_sum = pl.pallas_call(
        _sc_loss_body,
        out_specs=pl.BlockSpec(memory_space=pltpu.SMEM),
        out_shape=jax.ShapeDtypeStruct((1, 1), jnp.float32),
    )(pos[:s_rows], topk)
    return (tc_sum[0, 0] + sc_sum[0, 0]) / (b * k)


# speedup vs baseline: 2.2445x; 2.2445x over previous
import math

import jax
import jax.numpy as jnp
from jax.experimental import pallas as pl
from jax.experimental.pallas import tpu as pltpu

_K = 10
_POS_PENALTY = 1.2
_ROW_BLOCK = 256
_W = [1.0 / math.log2(j + 2.0) for j in range(_K)]
_NEG_BIG = float(-3.4028235e38)

_NET8 = [
    (0, 1), (2, 3), (4, 5), (6, 7),
    (0, 2), (1, 3), (4, 6), (5, 7),
    (1, 2), (5, 6), (0, 4), (1, 5),
    (2, 6), (3, 7), (2, 4), (3, 5),
    (1, 2), (3, 4), (5, 6),
]


def _pad_lanes(x):
    r, n = x.shape
    c = ((n + 127) // 128) * 128
    if c == n:
        return x
    return jnp.concatenate(
        [x, jnp.full((r, c - n), _NEG_BIG, jnp.float32)], axis=1
    )


def _loss_body(pos_ref, neg_hbm, out_ref, buf, sem):
    i = pl.program_id(0)
    nsteps = pl.num_programs(0)

    def start(step, slot):
        pltpu.make_async_copy(
            neg_hbm.at[:, pl.ds(step * _ROW_BLOCK, _ROW_BLOCK)],
            buf.at[slot], sem.at[slot],
        ).start()

    @pl.when(i == 0)
    def _():
        start(0, 0)

    @pl.when(i + 1 < nsteps)
    def _():
        start(i + 1, (i + 1) % 2)

    slot = i % 2
    pltpu.make_async_copy(
        neg_hbm.at[:, pl.ds(i * _ROW_BLOCK, _ROW_BLOCK)],
        buf.at[slot], sem.at[slot],
    ).wait()
    x = _pad_lanes(jnp.transpose(buf[slot]))
    pos = pos_ref[...]
    r, c = x.shape
    nq = c // 128
    lane = jax.lax.broadcasted_iota(jnp.int32, (r, 128), 1)
    xs = [x[:, i * 128:(i + 1) * 128] for i in range(nq)]
    s = []
    for i in range(nq):
        bits = jax.lax.bitcast_convert_type(xs[i], jnp.int32)
        pk = (bits & ~jnp.int32(1023)) | ((c - 1 - i * 128) - lane)
        s.append(jax.lax.bitcast_convert_type(pk, jnp.float32))
    for a, b in _NET8:
        hi = jnp.maximum(s[a], s[b])
        lo = jnp.minimum(s[a], s[b])
        s[a], s[b] = hi, lo
    tm = jnp.full((r, 128), _NEG_BIG, jnp.float32)
    m0p = None
    for j in range(_K):
        km = jnp.max(s[0], axis=1, keepdims=True)
        if j == 0:
            m0p = km
        qmax = min(nq - 1, _K - 1 - j)
        if qmax > 0:
            pop = s[0] == km
            for q in range(qmax):
                s[q] = jnp.where(pop, s[q + 1], s[q])
            if j + 8 < _K:
                s[nq - 1] = jnp.where(pop, _NEG_BIG, s[nq - 1])
        tm = jnp.where(lane == j, km, tm)
    tb = jax.lax.bitcast_convert_type(tm, jnp.int32) & ~jnp.int32(1023)
    tm = jax.lax.bitcast_convert_type(tb, jnp.float32)
    m0b = jax.lax.bitcast_convert_type(m0p, jnp.int32) & ~jnp.int32(1023)
    m0 = jax.lax.bitcast_convert_type(m0b, jnp.float32)
    wv = jnp.zeros((1, 128), jnp.float32)
    lane1 = jax.lax.broadcasted_iota(jnp.int32, (1, 128), 1)
    for j in range(_K):
        wv = wv + _W[j] * jnp.where(lane1 == j, 1.0, 0.0)
    z = tm - pos
    sp = jnp.maximum(z, 0.0) + jnp.log1p(jnp.exp(-jnp.abs(z)))
    pen = jnp.where(pos < m0, _POS_PENALTY, 1.0)
    blk = jnp.sum(jnp.sum(sp * wv, axis=1, keepdims=True) * pen)

    @pl.when(pl.program_id(0) == 0)
    def _():
        out_ref[0, 0] = 0.0

    out_ref[0, 0] += blk


def kernel(pos, neg):
    if pos.ndim == 1:
        pos = pos[:, None]
    b, n = neg.shape
    k = min(_K, n)
    assert k == _K
    grid = b // _ROW_BLOCK
    total = pl.pallas_call(
        _loss_body,
        grid=(grid,),
        in_specs=[
            pl.BlockSpec((_ROW_BLOCK, 1), lambda i: (i, 0)),
            pl.BlockSpec(memory_space=pl.ANY),
        ],
        out_specs=pl.BlockSpec(memory_space=pltpu.SMEM),
        out_shape=jax.ShapeDtypeStruct((1, 1), jnp.float32),
        scratch_shapes=[
            pltpu.VMEM((2, n, _ROW_BLOCK), jnp.float32),
            pltpu.SemaphoreType.DMA((2,)),
        ],
    )(pos, jnp.transpose(neg))
    return total[0, 0] / (b * k)
